# Initial kernel scaffold; baseline (speedup 1.0000x reference)
import jax
import jax.numpy as jnp
from jax.experimental import pallas as pl


def kernel(proj_feats, h_ctc_0, h_ctc_1, A_0, A_1, spikes_0, spikes_1, W_mem, b_mem, W_kv_0, b_kv_0, W_kv_1, b_kv_1, W_q, b_q, W_o, b_o, in_proj_w, in_proj_b, out_proj_w, out_proj_b, tags):
    raise NotImplementedError("write your pallas kernel here")


# speedup vs baseline: 1.5305x; 1.5305x over previous
import functools

import jax
import jax.numpy as jnp
from jax.experimental import pallas as pl
from jax.experimental.pallas import tpu as pltpu

B = 4
T = 2048
D_PROJ = 1024
D_C = 512
D_MODEL = 1024
N_HEADS = 16
HD = D_MODEL // N_HEADS
S0 = 64
GATE_R = 8
PER_SPK = 32
SIGMA = 4.0

TB = 256
NT = T // TB

_DNT = (((1,), (1,)), ((), ()))
F32 = jnp.float32


def _dott(a, b):
    return jax.lax.dot_general(a, b, _DNT, preferred_element_type=F32)


def _dot(a, b):
    return jax.lax.dot_general(a, b, (((1,), (0,)), ((), ())),
                               preferred_element_type=F32)


def _prep_kernel(h_ref, a_ref, sp_ref, wkv_ref, bkv_ref, wq_ref, bq_ref,
                 wqin_ref, bqin_ref, q_out, g_out):
    a_row = a_ref[0]
    s_row = sp_ref[0]
    s_col = jnp.transpose(s_row)

    t_row = jax.lax.broadcasted_iota(jnp.int32, (S0, T), 1)
    dist = t_row - s_col

    acc = jnp.zeros((S0, 1), F32)
    cnt = jnp.zeros((S0, 1), jnp.int32)
    for off in range(-GATE_R, GATE_R + 1):
        m = dist == off
        tap = jnp.sum(jnp.where(m, a_row, 0.0), axis=1, keepdims=True)
        acc = acc + tap
        idx = s_col + off
        cnt = cnt + ((idx >= 0) & (idx < T)).astype(jnp.int32)
    scores = acc / jnp.maximum(cnt, 1).astype(F32)
    scores_row = jnp.transpose(scores)

    ii = jax.lax.broadcasted_iota(jnp.int32, (S0, S0), 0)
    jj = jax.lax.broadcasted_iota(jnp.int32, (S0, S0), 1)
    gt = (scores_row > scores).astype(jnp.int32)
    eq = ((scores_row == scores) & (jj < ii)).astype(jnp.int32)
    rank = jnp.sum(gt + eq, axis=1, keepdims=True)
    rank_row = jnp.transpose(rank)

    r_col = jax.lax.broadcasted_iota(jnp.int32, (PER_SPK, 1), 0)
    sel = (rank_row == r_col).astype(jnp.int32)
    p = jnp.sum(sel * s_row, axis=1, keepdims=True)
    conf = jnp.sum(sel.astype(F32) * scores_row, axis=1, keepdims=True)

    g_out[0] = jax.nn.sigmoid(2.0 * conf)

    t2 = jax.lax.broadcasted_iota(jnp.int32, (PER_SPK, T), 1)
    d2 = t2 - p
    win = (d2 >= -GATE_R) & (d2 <= GATE_R)
    df = d2.astype(F32) * (1.0 / SIGMA)
    w = jnp.where(win, jnp.exp(-0.5 * df * df) * a_row, 0.0)
    wsum = jnp.sum(w, axis=1, keepdims=True)
    wn = w / (wsum + 1e-6)
    z = _dot(wn, h_ref[0])

    k_seed = _dott(z, wkv_ref[0]) + bkv_ref[...]
    qk = jnp.tanh(_dott(k_seed, wq_ref[...]) + bq_ref[...])
    q_out[0] = _dott(qk, wqin_ref[...]) + bqin_ref[...]


def _prep(h, a, sp, wkv1, bkv1, wq, bq, wqin, bqin):
    q, g = pl.pallas_call(
        _prep_kernel,
        grid=(B,),
        in_specs=[
            pl.BlockSpec((1, T, D_C), lambda b: (b, 0, 0)),
            pl.BlockSpec((1, 1, T), lambda b: (b, 0, 0)),
            pl.BlockSpec((1, 1, S0), lambda b: (b, 0, 0)),
            pl.BlockSpec((1, D_MODEL, D_C), lambda b: (0, 0, 0)),
            pl.BlockSpec((1, D_MODEL), lambda b: (0, 0)),
            pl.BlockSpec((D_MODEL, D_MODEL), lambda b: (0, 0)),
            pl.BlockSpec((1, D_MODEL), lambda b: (0, 0)),
            pl.BlockSpec((D_MODEL, D_MODEL), lambda b: (0, 0)),
            pl.BlockSpec((1, D_MODEL), lambda b: (0, 0)),
        ],
        out_specs=[
            pl.BlockSpec((1, PER_SPK, D_MODEL), lambda b: (b, 0, 0)),
            pl.BlockSpec((1, PER_SPK, 1), lambda b: (b, 0, 0)),
        ],
        out_shape=[
            jax.ShapeDtypeStruct((B, PER_SPK, D_MODEL), F32),
            jax.ShapeDtypeStruct((B, PER_SPK, 1), F32),
        ],
    )(h, a, sp, wkv1, bkv1, wq, bq, wqin, bqin)
    return q, g


def _wfuse_kernel(wk_ref, wv_ref, wmem_ref, bmem_ref, bk_ref, bv_ref,
                  wkf_out, wvf_out, bkf_out, bvf_out):
    wmem = wmem_ref[...]
    wkf_out[...] = _dot(wk_ref[...], wmem)
    wvf_out[...] = _dot(wv_ref[...], wmem)
    bmem = bmem_ref[...]
    bkf_out[...] = _dott(bmem, wk_ref[...]) + bk_ref[...]
    bvf_out[...] = _dott(bmem, wv_ref[...]) + bv_ref[...]


def _wfuse(wk, wv, wmem, bmem, bk, bv):
    full = lambda shape: pl.BlockSpec(shape, lambda: tuple(0 for _ in shape))
    return pl.pallas_call(
        _wfuse_kernel,
        in_specs=[full((D_MODEL, D_MODEL)), full((D_MODEL, D_MODEL)),
                  full((D_MODEL, D_PROJ)), full((1, D_MODEL)),
                  full((1, D_MODEL)), full((1, D_MODEL))],
        out_specs=[full((D_MODEL, D_PROJ)), full((D_MODEL, D_PROJ)),
                   full((1, D_MODEL)), full((1, D_MODEL))],
        out_shape=[
            jax.ShapeDtypeStruct((D_MODEL, D_PROJ), F32),
            jax.ShapeDtypeStruct((D_MODEL, D_PROJ), F32),
            jax.ShapeDtypeStruct((1, D_MODEL), F32),
            jax.ShapeDtypeStruct((1, D_MODEL), F32),
        ],
    )(wk, wv, wmem, bmem, bk, bv)


def _attn_kernel(pf_ref, wkf_ref, bkf_ref, wvf_ref, bvf_ref, q_ref, o_ref,
                 sc_s, v_s):
    tb = pl.program_id(1)
    x = pf_ref[0]
    kb = _dott(x, wkf_ref[...]) + bkf_ref[...]
    vb = _dott(x, wvf_ref[...]) + bvf_ref[...]
    v_s[tb] = vb
    qa = q_ref[0]
    for h in range(N_HEADS):
        qh = qa[:, h * HD:(h + 1) * HD]
        kh = kb[:, h * HD:(h + 1) * HD]
        sc_s[h, tb] = _dott(qh, kh) * (1.0 / (HD ** 0.5))

    @pl.when(tb == NT - 1)
    def _():
        for h in range(N_HEADS):
            sc = sc_s[h]
            m = jnp.max(jnp.max(sc, axis=2, keepdims=True), axis=0,
                        keepdims=True)
            e = jnp.exp(sc - m)
            den = jnp.sum(jnp.sum(e, axis=2, keepdims=True), axis=0,
                          keepdims=True)
            pmat = e / den
            acc = jnp.zeros((2 * PER_SPK, HD), F32)
            for t2 in range(NT):
                acc = acc + _dot(pmat[t2], v_s[t2, :, h * HD:(h + 1) * HD])
            o_ref[0, :, h * HD:(h + 1) * HD] = acc


def _attn(pf, wkf, bkf, wvf, bvf, q_all):
    sq = 2 * PER_SPK
    return pl.pallas_call(
        _attn_kernel,
        grid=(B, NT),
        in_specs=[
            pl.BlockSpec((1, TB, D_PROJ), lambda b, t: (b, t, 0)),
            pl.BlockSpec((D_MODEL, D_PROJ), lambda b, t: (0, 0)),
            pl.BlockSpec((1, D_MODEL), lambda b, t: (0, 0)),
            pl.BlockSpec((D_MODEL, D_PROJ), lambda b, t: (0, 0)),
            pl.BlockSpec((1, D_MODEL), lambda b, t: (0, 0)),
            pl.BlockSpec((1, sq, D_MODEL), lambda b, t: (b, 0, 0)),
        ],
        out_specs=pl.BlockSpec((1, sq, D_MODEL), lambda b, t: (b, 0, 0)),
        out_shape=jax.ShapeDtypeStruct((B, sq, D_MODEL), F32),
        scratch_shapes=[
            pltpu.VMEM((N_HEADS, NT, sq, TB), F32),
            pltpu.VMEM((NT, TB, D_MODEL), F32),
        ],
    )(pf, wkf, bkf, wvf, bvf, q_all)


def _out_kernel(o_ref, opw_ref, opb_ref, wo_ref, bo_ref, g_ref, a0_ref,
                a1_ref, tags_ref, out_ref):
    o = o_ref[0]
    f = _dott(o, opw_ref[...]) + opb_ref[...]
    f = _dott(f, wo_ref[...]) + bo_ref[...]
    g = g_ref[0]
    a0 = a0_ref[0, :, 0:1]
    a1 = a1_ref[0, :, 0:1]
    den = a0 + a1 + 1e-6
    tags = tags_ref[...]
    slot = (a0 / den) * tags[0:1, :] + (a1 / den) * tags[1:2, :]
    out_ref[0] = f * g + slot


def _out(o, opw, opb, wo, bo, g_all, a0s, a1s, tags):
    sq = 2 * PER_SPK
    stride = T // sq
    return pl.pallas_call(
        _out_kernel,
        grid=(B,),
        in_specs=[
            pl.BlockSpec((1, sq, D_MODEL), lambda b: (b, 0, 0)),
            pl.BlockSpec((D_MODEL, D_MODEL), lambda b: (0, 0)),
            pl.BlockSpec((1, D_MODEL), lambda b: (0, 0)),
            pl.BlockSpec((D_MODEL, D_MODEL), lambda b: (0, 0)),
            pl.BlockSpec((1, D_MODEL), lambda b: (0, 0)),
            pl.BlockSpec((1, sq, 1), lambda b: (b, 0, 0)),
            pl.BlockSpec((1, sq, stride), lambda b: (b, 0, 0)),
            pl.BlockSpec((1, sq, stride), lambda b: (b, 0, 0)),
            pl.BlockSpec((2, D_MODEL), lambda b: (0, 0)),
        ],
        out_specs=pl.BlockSpec((1, sq, D_MODEL), lambda b: (b, 0, 0)),
        out_shape=jax.ShapeDtypeStruct((B, sq, D_MODEL), F32),
    )(o, opw, opb, wo, bo, g_all, a0s, a1s, tags)


def kernel(proj_feats, h_ctc_0, h_ctc_1, A_0, A_1, spikes_0, spikes_1,
           W_mem, b_mem, W_kv_0, b_kv_0, W_kv_1, b_kv_1, W_q, b_q, W_o, b_o,
           in_proj_w, in_proj_b, out_proj_w, out_proj_b, tags):
    wqi = in_proj_w[0:D_MODEL]
    wki = in_proj_w[D_MODEL:2 * D_MODEL]
    wvi = in_proj_w[2 * D_MODEL:3 * D_MODEL]
    bqi = in_proj_b[0:D_MODEL].reshape(1, D_MODEL)
    bki = in_proj_b[D_MODEL:2 * D_MODEL].reshape(1, D_MODEL)
    bvi = in_proj_b[2 * D_MODEL:3 * D_MODEL].reshape(1, D_MODEL)
    bq2 = b_q.reshape(1, D_MODEL)
    bo2 = b_o.reshape(1, D_MODEL)
    opb2 = out_proj_b.reshape(1, D_MODEL)
    bmem2 = b_mem.reshape(1, D_MODEL)

    q0, g0 = _prep(h_ctc_0, A_0.reshape(B, 1, T), spikes_0.reshape(B, 1, S0),
                   W_kv_0[:D_MODEL].reshape(1, D_MODEL, D_C),
                   b_kv_0[:D_MODEL].reshape(1, D_MODEL), W_q, bq2, wqi, bqi)
    q1, g1 = _prep(h_ctc_1, A_1.reshape(B, 1, T), spikes_1.reshape(B, 1, S0),
                   W_kv_1[:D_MODEL].reshape(1, D_MODEL, D_C),
                   b_kv_1[:D_MODEL].reshape(1, D_MODEL), W_q, bq2, wqi, bqi)
    q_all = jnp.concatenate([q0, q1], axis=1)
    g_all = jnp.concatenate([g0, g1], axis=1)

    wkf, wvf, bkf, bvf = _wfuse(wki, wvi, W_mem, bmem2, bki, bvi)
    o = _attn(proj_feats, wkf, bkf, wvf, bvf, q_all)

    sq = 2 * PER_SPK
    a0s = A_0.reshape(B, sq, T // sq)
    a1s = A_1.reshape(B, sq, T // sq)
    return _out(o, out_proj_w, opb2, W_o, bo2, g_all, a0s, a1s, tags)


# speedup vs baseline: 1.5325x; 1.0013x over previous
import functools

import jax
import jax.numpy as jnp
from jax.experimental import pallas as pl
from jax.experimental.pallas import tpu as pltpu

B = 4
T = 2048
D_PROJ = 1024
D_C = 512
D_MODEL = 1024
N_HEADS = 16
HD = D_MODEL // N_HEADS
S0 = 64
GATE_R = 8
PER_SPK = 32
SIGMA = 4.0

TB = 256
NT = T // TB

_DNT = (((1,), (1,)), ((), ()))
F32 = jnp.float32


def _dott(a, b):
    return jax.lax.dot_general(a, b, _DNT, preferred_element_type=F32)


def _dot(a, b):
    return jax.lax.dot_general(a, b, (((1,), (0,)), ((), ())),
                               preferred_element_type=F32)


def _prep_kernel(h_ref, a_ref, sp_ref, wkv_ref, bkv_ref, wq_ref, bq_ref,
                 wqin_ref, bqin_ref, q_out, g_out):
    a_row = a_ref[0]
    s_row = sp_ref[0]
    s_col = jnp.transpose(s_row)

    t_row = jax.lax.broadcasted_iota(jnp.int32, (S0, T), 1)
    dist = t_row - s_col

    acc = jnp.zeros((S0, 1), F32)
    cnt = jnp.zeros((S0, 1), jnp.int32)
    for off in range(-GATE_R, GATE_R + 1):
        m = dist == off
        tap = jnp.sum(jnp.where(m, a_row, 0.0), axis=1, keepdims=True)
        acc = acc + tap
        idx = s_col + off
        cnt = cnt + ((idx >= 0) & (idx < T)).astype(jnp.int32)
    scores = acc / jnp.maximum(cnt, 1).astype(F32)
    scores_row = jnp.transpose(scores)

    ii = jax.lax.broadcasted_iota(jnp.int32, (S0, S0), 0)
    jj = jax.lax.broadcasted_iota(jnp.int32, (S0, S0), 1)
    gt = (scores_row > scores).astype(jnp.int32)
    eq = ((scores_row == scores) & (jj < ii)).astype(jnp.int32)
    rank = jnp.sum(gt + eq, axis=1, keepdims=True)
    rank_row = jnp.transpose(rank)

    r_col = jax.lax.broadcasted_iota(jnp.int32, (PER_SPK, 1), 0)
    sel = (rank_row == r_col).astype(jnp.int32)
    p = jnp.sum(sel * s_row, axis=1, keepdims=True)
    conf = jnp.sum(sel.astype(F32) * scores_row, axis=1, keepdims=True)

    g_out[0] = jax.nn.sigmoid(2.0 * conf)

    t2 = jax.lax.broadcasted_iota(jnp.int32, (PER_SPK, T), 1)
    d2 = t2 - p
    win = (d2 >= -GATE_R) & (d2 <= GATE_R)
    df = d2.astype(F32) * (1.0 / SIGMA)
    w = jnp.where(win, jnp.exp(-0.5 * df * df) * a_row, 0.0)
    wsum = jnp.sum(w, axis=1, keepdims=True)
    wn = w / (wsum + 1e-6)
    z = _dot(wn, h_ref[0])

    k_seed = _dott(z, wkv_ref[0]) + bkv_ref[...]
    qk = jnp.tanh(_dott(k_seed, wq_ref[...]) + bq_ref[...])
    q_out[0] = _dott(qk, wqin_ref[...]) + bqin_ref[...]


def _prep(h, a, sp, wkv1, bkv1, wq, bq, wqin, bqin):
    q, g = pl.pallas_call(
        _prep_kernel,
        grid=(B,),
        in_specs=[
            pl.BlockSpec((1, T, D_C), lambda b: (b, 0, 0)),
            pl.BlockSpec((1, 1, T), lambda b: (b, 0, 0)),
            pl.BlockSpec((1, 1, S0), lambda b: (b, 0, 0)),
            pl.BlockSpec((1, D_MODEL, D_C), lambda b: (0, 0, 0)),
            pl.BlockSpec((1, D_MODEL), lambda b: (0, 0)),
            pl.BlockSpec((D_MODEL, D_MODEL), lambda b: (0, 0)),
            pl.BlockSpec((1, D_MODEL), lambda b: (0, 0)),
            pl.BlockSpec((D_MODEL, D_MODEL), lambda b: (0, 0)),
            pl.BlockSpec((1, D_MODEL), lambda b: (0, 0)),
        ],
        out_specs=[
            pl.BlockSpec((1, PER_SPK, D_MODEL), lambda b: (b, 0, 0)),
            pl.BlockSpec((1, PER_SPK, 1), lambda b: (b, 0, 0)),
        ],
        out_shape=[
            jax.ShapeDtypeStruct((B, PER_SPK, D_MODEL), F32),
            jax.ShapeDtypeStruct((B, PER_SPK, 1), F32),
        ],
    )(h, a, sp, wkv1, bkv1, wq, bq, wqin, bqin)
    return q, g


def _wfuse_kernel(wk_ref, wv_ref, wmem_ref, bmem_ref, bk_ref, bv_ref,
                  wkf_out, wvf_out, bkf_out, bvf_out):
    wmem = wmem_ref[...].astype(jnp.bfloat16)
    wkb = wk_ref[...].astype(jnp.bfloat16)
    wvb = wv_ref[...].astype(jnp.bfloat16)
    wkf_out[...] = _dot(wkb, wmem).astype(jnp.bfloat16)
    wvf_out[...] = _dot(wvb, wmem).astype(jnp.bfloat16)
    bmem = bmem_ref[...]
    bkf_out[...] = _dott(bmem, wk_ref[...]) + bk_ref[...]
    bvf_out[...] = _dott(bmem, wv_ref[...]) + bv_ref[...]


def _wfuse(wk, wv, wmem, bmem, bk, bv):
    full = lambda shape: pl.BlockSpec(shape, lambda: tuple(0 for _ in shape))
    return pl.pallas_call(
        _wfuse_kernel,
        in_specs=[full((D_MODEL, D_MODEL)), full((D_MODEL, D_MODEL)),
                  full((D_MODEL, D_PROJ)), full((1, D_MODEL)),
                  full((1, D_MODEL)), full((1, D_MODEL))],
        out_specs=[full((D_MODEL, D_PROJ)), full((D_MODEL, D_PROJ)),
                   full((1, D_MODEL)), full((1, D_MODEL))],
        out_shape=[
            jax.ShapeDtypeStruct((D_MODEL, D_PROJ), jnp.bfloat16),
            jax.ShapeDtypeStruct((D_MODEL, D_PROJ), jnp.bfloat16),
            jax.ShapeDtypeStruct((1, D_MODEL), F32),
            jax.ShapeDtypeStruct((1, D_MODEL), F32),
        ],
    )(wk, wv, wmem, bmem, bk, bv)


def _attn_kernel(pf_ref, wkf_ref, bkf_ref, wvf_ref, bvf_ref, q_ref, o_ref,
                 sc_s, v_s):
    tb = pl.program_id(1)
    x = pf_ref[0].astype(jnp.bfloat16)
    kb = _dott(x, wkf_ref[...]) + bkf_ref[...]
    vb = _dott(x, wvf_ref[...]) + bvf_ref[...]
    v_s[tb] = vb.astype(jnp.bfloat16)
    kbb = kb.astype(jnp.bfloat16)
    qa = q_ref[0].astype(jnp.bfloat16)
    for h in range(N_HEADS):
        qh = qa[:, h * HD:(h + 1) * HD]
        kh = kbb[:, h * HD:(h + 1) * HD]
        sc_s[h, tb] = _dott(qh, kh) * (1.0 / (HD ** 0.5))

    @pl.when(tb == NT - 1)
    def _():
        for h in range(N_HEADS):
            sc = sc_s[h]
            m = jnp.max(jnp.max(sc, axis=2, keepdims=True), axis=0,
                        keepdims=True)
            e = jnp.exp(sc - m)
            den = jnp.sum(jnp.sum(e, axis=2, keepdims=True), axis=0,
                          keepdims=True)
            pmat = (e / den).astype(jnp.bfloat16)
            acc = jnp.zeros((2 * PER_SPK, HD), F32)
            for t2 in range(NT):
                acc = acc + _dot(pmat[t2], v_s[t2, :, h * HD:(h + 1) * HD])
            o_ref[0, :, h * HD:(h + 1) * HD] = acc


def _attn(pf, wkf, bkf, wvf, bvf, q_all):
    sq = 2 * PER_SPK
    return pl.pallas_call(
        _attn_kernel,
        grid=(B, NT),
        in_specs=[
            pl.BlockSpec((1, TB, D_PROJ), lambda b, t: (b, t, 0)),
            pl.BlockSpec((D_MODEL, D_PROJ), lambda b, t: (0, 0)),
            pl.BlockSpec((1, D_MODEL), lambda b, t: (0, 0)),
            pl.BlockSpec((D_MODEL, D_PROJ), lambda b, t: (0, 0)),
            pl.BlockSpec((1, D_MODEL), lambda b, t: (0, 0)),
            pl.BlockSpec((1, sq, D_MODEL), lambda b, t: (b, 0, 0)),
        ],
        out_specs=pl.BlockSpec((1, sq, D_MODEL), lambda b, t: (b, 0, 0)),
        out_shape=jax.ShapeDtypeStruct((B, sq, D_MODEL), F32),
        scratch_shapes=[
            pltpu.VMEM((N_HEADS, NT, sq, TB), F32),
            pltpu.VMEM((NT, TB, D_MODEL), jnp.bfloat16),
        ],
    )(pf, wkf, bkf, wvf, bvf, q_all)


def _out_kernel(o_ref, opw_ref, opb_ref, wo_ref, bo_ref, g_ref, a0_ref,
                a1_ref, tags_ref, out_ref):
    o = o_ref[0]
    f = _dott(o, opw_ref[...]) + opb_ref[...]
    f = _dott(f, wo_ref[...]) + bo_ref[...]
    g = g_ref[0]
    a0 = a0_ref[0, :, 0:1]
    a1 = a1_ref[0, :, 0:1]
    den = a0 + a1 + 1e-6
    tags = tags_ref[...]
    slot = (a0 / den) * tags[0:1, :] + (a1 / den) * tags[1:2, :]
    out_ref[0] = f * g + slot


def _out(o, opw, opb, wo, bo, g_all, a0s, a1s, tags):
    sq = 2 * PER_SPK
    stride = T // sq
    return pl.pallas_call(
        _out_kernel,
        grid=(B,),
        in_specs=[
            pl.BlockSpec((1, sq, D_MODEL), lambda b: (b, 0, 0)),
            pl.BlockSpec((D_MODEL, D_MODEL), lambda b: (0, 0)),
            pl.BlockSpec((1, D_MODEL), lambda b: (0, 0)),
            pl.BlockSpec((D_MODEL, D_MODEL), lambda b: (0, 0)),
            pl.BlockSpec((1, D_MODEL), lambda b: (0, 0)),
            pl.BlockSpec((1, sq, 1), lambda b: (b, 0, 0)),
            pl.BlockSpec((1, sq, stride), lambda b: (b, 0, 0)),
            pl.BlockSpec((1, sq, stride), lambda b: (b, 0, 0)),
            pl.BlockSpec((2, D_MODEL), lambda b: (0, 0)),
        ],
        out_specs=pl.BlockSpec((1, sq, D_MODEL), lambda b: (b, 0, 0)),
        out_shape=jax.ShapeDtypeStruct((B, sq, D_MODEL), F32),
    )(o, opw, opb, wo, bo, g_all, a0s, a1s, tags)


def kernel(proj_feats, h_ctc_0, h_ctc_1, A_0, A_1, spikes_0, spikes_1,
           W_mem, b_mem, W_kv_0, b_kv_0, W_kv_1, b_kv_1, W_q, b_q, W_o, b_o,
           in_proj_w, in_proj_b, out_proj_w, out_proj_b, tags):
    wqi = in_proj_w[0:D_MODEL]
    wki = in_proj_w[D_MODEL:2 * D_MODEL]
    wvi = in_proj_w[2 * D_MODEL:3 * D_MODEL]
    bqi = in_proj_b[0:D_MODEL].reshape(1, D_MODEL)
    bki = in_proj_b[D_MODEL:2 * D_MODEL].reshape(1, D_MODEL)
    bvi = in_proj_b[2 * D_MODEL:3 * D_MODEL].reshape(1, D_MODEL)
    bq2 = b_q.reshape(1, D_MODEL)
    bo2 = b_o.reshape(1, D_MODEL)
    opb2 = out_proj_b.reshape(1, D_MODEL)
    bmem2 = b_mem.reshape(1, D_MODEL)

    q0, g0 = _prep(h_ctc_0, A_0.reshape(B, 1, T), spikes_0.reshape(B, 1, S0),
                   W_kv_0[:D_MODEL].reshape(1, D_MODEL, D_C),
                   b_kv_0[:D_MODEL].reshape(1, D_MODEL), W_q, bq2, wqi, bqi)
    q1, g1 = _prep(h_ctc_1, A_1.reshape(B, 1, T), spikes_1.reshape(B, 1, S0),
                   W_kv_1[:D_MODEL].reshape(1, D_MODEL, D_C),
                   b_kv_1[:D_MODEL].reshape(1, D_MODEL), W_q, bq2, wqi, bqi)
    q_all = jnp.concatenate([q0, q1], axis=1)
    g_all = jnp.concatenate([g0, g1], axis=1)

    wkf, wvf, bkf, bvf = _wfuse(wki, wvi, W_mem, bmem2, bki, bvi)
    o = _attn(proj_feats, wkf, bkf, wvf, bvf, q_all)

    sq = 2 * PER_SPK
    a0s = A_0.reshape(B, sq, T // sq)
    a1s = A_1.reshape(B, sq, T // sq)
    return _out(o, out_proj_w, opb2, W_o, bo2, g_all, a0s, a1s, tags)
